# baseline (speedup 1.0000x reference)
import jax
import jax.numpy as jnp
from jax.experimental import pallas as pl
from jax.experimental.pallas import tpu as pltpu

_H = 768
_E = 64


def _router_body(x_hbm, w1_hbm, b1_hbm, w2_hbm, b2_hbm, out_ref,
                 x_s, w1_s, b1_s, w2_s, b2_s,
                 sem0, sem1, sem2, sem3, sem4):
    cx = pltpu.make_async_copy(x_hbm.at[pl.ds(0, 1)], x_s, sem0)
    cw1 = pltpu.make_async_copy(w1_hbm, w1_s, sem1)
    cb1 = pltpu.make_async_copy(b1_hbm, b1_s, sem2)
    cw2 = pltpu.make_async_copy(w2_hbm, w2_s, sem3)
    cb2 = pltpu.make_async_copy(b2_hbm, b2_s, sem4)
    cx.start()
    cw1.start()
    cb1.start()
    cw2.start()
    cb2.start()

    cx.wait()
    cw1.wait()
    h = jax.lax.dot_general(
        x_s[...], w1_s[...], (((1,), (1,)), ((), ())),
        preferred_element_type=jnp.float32)
    cb1.wait()
    h = jnp.maximum(h + b1_s[...], 0.0)
    cw2.wait()
    logits = jax.lax.dot_general(
        h, w2_s[...], (((1,), (1,)), ((), ())),
        preferred_element_type=jnp.float32)
    cb2.wait()
    logits = logits + b2_s[...]

    ids = jax.lax.broadcasted_iota(jnp.int32, (1, _E), 1)
    ninf = jnp.float32(-jnp.inf)
    v1 = jnp.max(logits, axis=1, keepdims=True)
    t1 = logits == v1
    i1 = jnp.min(jnp.where(t1, ids, _E), axis=1, keepdims=True)
    r2 = jnp.max(jnp.where(t1, ninf, logits), axis=1, keepdims=True)
    cnt = jnp.sum(t1.astype(jnp.float32), axis=1, keepdims=True)
    dup = cnt >= 2.0
    v2 = jnp.where(dup, v1, r2)
    i2a = jnp.min(jnp.where(t1 & (ids > i1), ids, _E), axis=1, keepdims=True)
    i2b = jnp.min(jnp.where(logits == r2, ids, _E), axis=1, keepdims=True)
    i2 = jnp.where(dup, i2a, i2b)

    e2 = jnp.exp(v2 - v1)
    denom = 1.0 + e2
    out_ref[...] = jnp.where(
        ids == i1, 1.0 / denom, jnp.where(ids == i2, e2 / denom, 0.0))


def kernel(x, W1, b1, W2, b2):
    out = pl.pallas_call(
        _router_body,
        grid=(1,),
        in_specs=[
            pl.BlockSpec(memory_space=pl.ANY),
            pl.BlockSpec(memory_space=pl.ANY),
            pl.BlockSpec(memory_space=pl.ANY),
            pl.BlockSpec(memory_space=pl.ANY),
            pl.BlockSpec(memory_space=pl.ANY),
        ],
        out_specs=pl.BlockSpec((1, _E), lambda i: (0, 0)),
        out_shape=jax.ShapeDtypeStruct((1, _E), jnp.float32),
        scratch_shapes=[
            pltpu.VMEM((1, _H), jnp.float32),
            pltpu.VMEM((_H, _H), jnp.float32),
            pltpu.VMEM((1, _H), jnp.float32),
            pltpu.VMEM((_E, _H), jnp.float32),
            pltpu.VMEM((1, _E), jnp.float32),
            pltpu.SemaphoreType.DMA,
            pltpu.SemaphoreType.DMA,
            pltpu.SemaphoreType.DMA,
            pltpu.SemaphoreType.DMA,
            pltpu.SemaphoreType.DMA,
        ],
    )(x.reshape(32768, _H), W1, b1.reshape(1, _H), W2, b2.reshape(1, _E))
    return out.reshape(_E)


# speedup vs baseline: 19.2495x; 19.2495x over previous
import jax
import jax.numpy as jnp
from jax.experimental import pallas as pl
from jax.experimental.pallas import tpu as pltpu

_H = 768
_E = 64


def _router_body(x_hbm, w1_hbm, b1_hbm, w2_hbm, b2_hbm, out_ref,
                 x_s, w1_s, b1_s, w2_s, b2_s,
                 sem0, sem1, sem2, sem3, sem4):
    cx = pltpu.make_async_copy(x_hbm.at[pl.ds(0, 1)], x_s, sem0)
    cw1 = pltpu.make_async_copy(w1_hbm, w1_s, sem1)
    cb1 = pltpu.make_async_copy(b1_hbm, b1_s, sem2)
    cw2 = pltpu.make_async_copy(w2_hbm, w2_s, sem3)
    cb2 = pltpu.make_async_copy(b2_hbm, b2_s, sem4)
    cx.start()
    cw1.start()
    cb1.start()
    cw2.start()
    cb2.start()

    cx.wait()
    cw1.wait()
    h = jax.lax.dot_general(
        x_s[0], w1_s[...], (((1,), (1,)), ((), ())),
        preferred_element_type=jnp.float32)
    cb1.wait()
    h = jnp.maximum(h + b1_s[...], 0.0)
    cw2.wait()
    logits = jax.lax.dot_general(
        h, w2_s[...], (((1,), (1,)), ((), ())),
        preferred_element_type=jnp.float32)
    cb2.wait()
    logits = logits + b2_s[...]

    ids = jax.lax.broadcasted_iota(jnp.int32, (1, _E), 1)
    ninf = jnp.float32(-jnp.inf)
    v1 = jnp.max(logits, axis=1, keepdims=True)
    t1 = logits == v1
    i1 = jnp.min(jnp.where(t1, ids, _E), axis=1, keepdims=True)
    r2 = jnp.max(jnp.where(t1, ninf, logits), axis=1, keepdims=True)
    cnt = jnp.sum(t1.astype(jnp.float32), axis=1, keepdims=True)
    dup = cnt >= 2.0
    v2 = jnp.where(dup, v1, r2)
    i2a = jnp.min(jnp.where(t1 & (ids > i1), ids, _E), axis=1, keepdims=True)
    i2b = jnp.min(jnp.where(logits == r2, ids, _E), axis=1, keepdims=True)
    i2 = jnp.where(dup, i2a, i2b)

    e2 = jnp.exp(v2 - v1)
    denom = 1.0 + e2
    out_ref[...] = jnp.where(
        ids == i1, 1.0 / denom, jnp.where(ids == i2, e2 / denom, 0.0))


def kernel(x, W1, b1, W2, b2):
    out = pl.pallas_call(
        _router_body,
        grid=(1,),
        in_specs=[
            pl.BlockSpec(memory_space=pl.ANY),
            pl.BlockSpec(memory_space=pl.ANY),
            pl.BlockSpec(memory_space=pl.ANY),
            pl.BlockSpec(memory_space=pl.ANY),
            pl.BlockSpec(memory_space=pl.ANY),
        ],
        out_specs=pl.BlockSpec((1, _E), lambda i: (0, 0)),
        out_shape=jax.ShapeDtypeStruct((1, _E), jnp.float32),
        scratch_shapes=[
            pltpu.VMEM((1, 1, _H), jnp.float32),
            pltpu.VMEM((_H, _H), jnp.float32),
            pltpu.VMEM((1, _H), jnp.float32),
            pltpu.VMEM((_E, _H), jnp.float32),
            pltpu.VMEM((1, _E), jnp.float32),
            pltpu.SemaphoreType.DMA,
            pltpu.SemaphoreType.DMA,
            pltpu.SemaphoreType.DMA,
            pltpu.SemaphoreType.DMA,
            pltpu.SemaphoreType.DMA,
        ],
    )(x, W1, b1.reshape(1, _H), W2, b2.reshape(1, _E))
    return out.reshape(_E)


# speedup vs baseline: 20.2320x; 1.0510x over previous
import jax
import jax.numpy as jnp
from jax.experimental import pallas as pl

_H = 768
_E = 64


def _router_body(x_ref, w1_ref, b1_ref, w2_ref, b2_ref, out_ref):
    x0 = x_ref[0]
    h = jax.lax.dot_general(
        x0, w1_ref[...], (((1,), (1,)), ((), ())),
        preferred_element_type=jnp.float32)
    h = jnp.maximum(h + b1_ref[...], 0.0)
    logits = jax.lax.dot_general(
        h, w2_ref[...], (((1,), (1,)), ((), ())),
        preferred_element_type=jnp.float32)
    logits = logits + b2_ref[...]

    ids = jax.lax.broadcasted_iota(jnp.int32, (1, _E), 1)
    ninf = jnp.float32(-jnp.inf)
    v1 = jnp.max(logits, axis=1, keepdims=True)
    t1 = logits == v1
    i1 = jnp.min(jnp.where(t1, ids, _E), axis=1, keepdims=True)
    r2 = jnp.max(jnp.where(t1, ninf, logits), axis=1, keepdims=True)
    cnt = jnp.sum(t1.astype(jnp.float32), axis=1, keepdims=True)
    dup = cnt >= 2.0
    v2 = jnp.where(dup, v1, r2)
    i2a = jnp.min(jnp.where(t1 & (ids > i1), ids, _E), axis=1, keepdims=True)
    i2b = jnp.min(jnp.where(logits == r2, ids, _E), axis=1, keepdims=True)
    i2 = jnp.where(dup, i2a, i2b)

    e2 = jnp.exp(v2 - v1)
    denom = 1.0 + e2
    out_ref[...] = jnp.where(
        ids == i1, 1.0 / denom, jnp.where(ids == i2, e2 / denom, 0.0))


def kernel(x, W1, b1, W2, b2):
    out = pl.pallas_call(
        _router_body,
        grid=(1,),
        in_specs=[
            pl.BlockSpec((1, 1, _H), lambda i: (0, 0, 0)),
            pl.BlockSpec((_H, _H), lambda i: (0, 0)),
            pl.BlockSpec((1, _H), lambda i: (0, 0)),
            pl.BlockSpec((_E, _H), lambda i: (0, 0)),
            pl.BlockSpec((1, _E), lambda i: (0, 0)),
        ],
        out_specs=pl.BlockSpec((1, _E), lambda i: (0, 0)),
        out_shape=jax.ShapeDtypeStruct((1, _E), jnp.float32),
    )(x, W1, b1.reshape(1, _H), W2, b2.reshape(1, _E))
    return out.reshape(_E)
